# baseline (speedup 1.0000x reference)
import functools

import jax
import jax.numpy as jnp
from jax import lax
from jax.experimental import pallas as pl
from jax.experimental.pallas import tpu as pltpu
from jax.experimental.pallas import tpu_sc as plsc


def _make_gather(F, V, D, N):
    info = plsc.get_sparse_core_info()
    NC, NS, L = info.num_cores, info.num_subcores, info.num_lanes
    NW = NC * NS
    assert N % NW == 0
    per_w = N // NW
    CHUNK = 1664
    ROWS = CHUNK // 128
    assert per_w % CHUNK == 0
    n_chunks = per_w // CHUNK

    mesh = plsc.VectorSubcoreMesh(core_axis_name="c", subcore_axis_name="s")

    @functools.partial(
        pl.kernel,
        mesh=mesh,
        out_type=jax.ShapeDtypeStruct((N, D), jnp.float32),
        compiler_params=pltpu.CompilerParams(use_tc_tiling_on_sc=False),
        scratch_types=[
            pltpu.VMEM((CHUNK,), jnp.int32),
            pltpu.VMEM((CHUNK,), jnp.int32),
            pltpu.VMEM((CHUNK, D), jnp.float32),
            pltpu.SemaphoreType.DMA,
        ],
    )
    def gather_kernel(tab_hbm, idx_hbm, out_hbm, idx_v, off_v, rows_v, sem):
        wid = lax.axis_index("s") * NC + lax.axis_index("c")
        base_w = wid * per_w

        lanes = lax.iota(jnp.int32, L)
        for k in range(CHUNK // L):
            g = lanes + k * L
            off_v[pl.ds(k * L, L)] = (g % F) * V

        def do_chunk(ci, carry):
            base = base_w + ci * CHUNK
            pltpu.sync_copy(idx_hbm.at[pl.ds(base, CHUNK)], idx_v)
            for k in range(CHUNK // L):
                s = pl.ds(k * L, L)
                idx_v[s] = idx_v[s] + off_v[s]
            copies = []
            for r in range(ROWS):
                copies.append(
                    pltpu.async_copy(
                        tab_hbm.at[idx_v.at[pl.ds(r * 128, 128)]],
                        rows_v.at[pl.ds(r * 128, 128)],
                        sem,
                    )
                )
            for cp in copies:
                cp.wait()
            pltpu.sync_copy(rows_v, out_hbm.at[pl.ds(base, CHUNK)])
            return carry

        lax.fori_loop(0, n_chunks, do_chunk, 0)

    return gather_kernel


def kernel(inputs, tables):
    B, F = inputs.shape
    _, V, D = tables.shape
    N = B * F
    tab_flat = tables.reshape(F * V, D)
    idx_flat = inputs.reshape(N)
    out = _make_gather(F, V, D, N)(tab_flat, idx_flat)
    return out.reshape(B, F, D)


# speedup vs baseline: 1.0008x; 1.0008x over previous
import functools

import jax
import jax.numpy as jnp
from jax import lax
from jax.experimental import pallas as pl
from jax.experimental.pallas import tpu as pltpu
from jax.experimental.pallas import tpu_sc as plsc


def _make_gather(F, V, D, N):
    info = plsc.get_sparse_core_info()
    NC, NS, L = info.num_cores, info.num_subcores, info.num_lanes
    NW = NC * NS
    assert N % NW == 0
    per_w = N // NW
    CHUNK = 1664
    ROWS = CHUNK // 128
    PERIOD = 13 * L
    assert per_w % CHUNK == 0 and per_w % PERIOD == 0
    n_chunks = per_w // CHUNK

    mesh = plsc.VectorSubcoreMesh(core_axis_name="c", subcore_axis_name="s")

    @functools.partial(
        pl.kernel,
        mesh=mesh,
        out_type=jax.ShapeDtypeStruct((N, D), jnp.float32),
        compiler_params=pltpu.CompilerParams(use_tc_tiling_on_sc=False),
        scratch_types=[
            pltpu.VMEM((per_w,), jnp.int32),
            pltpu.VMEM((PERIOD,), jnp.int32),
            pltpu.VMEM((CHUNK, D), jnp.float32),
            pltpu.VMEM((CHUNK, D), jnp.float32),
            pltpu.SemaphoreType.DMA,
            pltpu.SemaphoreType.DMA,
            pltpu.SemaphoreType.DMA,
        ],
    )
    def gather_kernel(tab_hbm, idx_hbm, out_hbm, idx_v, off_v, rows0, rows1,
                      gsem, osem0, osem1):
        wid = lax.axis_index("s") * NC + lax.axis_index("c")
        base_w = wid * per_w

        pltpu.sync_copy(idx_hbm.at[pl.ds(base_w, per_w)], idx_v)

        lanes = lax.iota(jnp.int32, L)
        for r in range(PERIOD // L):
            off_v[pl.ds(r * L, L)] = ((lanes + r * L) % F) * V

        def add_block(j, carry):
            b = j * PERIOD
            for r in range(PERIOD // L):
                s = pl.ds(b + r * L, L)
                idx_v[s] = idx_v[s] + off_v[pl.ds(r * L, L)]
            return carry

        lax.fori_loop(0, per_w // PERIOD, add_block, 0)

        rows = [rows0, rows1]
        osems = [osem0, osem1]
        out_cps = [None, None]
        for c in range(n_chunks):
            p = c & 1
            if out_cps[p] is not None:
                out_cps[p].wait()
            cbase = c * CHUNK
            g_cps = [
                pltpu.async_copy(
                    tab_hbm.at[idx_v.at[pl.ds(cbase + r * 128, 128)]],
                    rows[p].at[pl.ds(r * 128, 128)],
                    gsem,
                )
                for r in range(ROWS)
            ]
            for cp in g_cps:
                cp.wait()
            out_cps[p] = pltpu.async_copy(
                rows[p], out_hbm.at[pl.ds(base_w + cbase, CHUNK)], osems[p]
            )
        for cp in out_cps:
            if cp is not None:
                cp.wait()

    return gather_kernel


def kernel(inputs, tables):
    B, F = inputs.shape
    _, V, D = tables.shape
    N = B * F
    tab_flat = tables.reshape(F * V, D)
    idx_flat = inputs.reshape(N)
    out = _make_gather(F, V, D, N)(tab_flat, idx_flat)
    return out.reshape(B, F, D)


# speedup vs baseline: 1.5448x; 1.5435x over previous
import functools

import jax
import jax.numpy as jnp
from jax import lax
from jax.experimental import pallas as pl
from jax.experimental.pallas import tpu as pltpu
from jax.experimental.pallas import tpu_sc as plsc

_info = plsc.get_sparse_core_info()
_NC, _NS, _L = _info.num_cores, _info.num_subcores, _info.num_lanes
_NW = _NC * _NS


def _make_gather_dmajor(F, V, D, B):
    R = F * D
    assert R % _NW == 0
    rows_per_w = R // _NW
    BCHUNK = 2048
    n_chunks = B // BCHUNK
    UNROLL = 8
    mesh = plsc.VectorSubcoreMesh(core_axis_name="c", subcore_axis_name="s")

    @functools.partial(
        pl.kernel,
        mesh=mesh,
        out_type=jax.ShapeDtypeStruct((R * B,), jnp.float32),
        compiler_params=pltpu.CompilerParams(
            use_tc_tiling_on_sc=False, needs_layout_passes=False
        ),
        scratch_types=[
            pltpu.VMEM((V,), jnp.float32),
            pltpu.VMEM((BCHUNK,), jnp.int32),
            pltpu.VMEM((BCHUNK,), jnp.float32),
            pltpu.SemaphoreType.DMA,
        ],
    )
    def gather_kernel(tab_hbm, idx_hbm, out_hbm, slab, idxb, outb, sem):
        wid = lax.axis_index("s") * _NC + lax.axis_index("c")
        row0 = wid * rows_per_w

        def do_row(t, carry):
            row = row0 + t
            f = row // D
            pltpu.sync_copy(tab_hbm.at[row], slab)

            def do_chunk(cb, carry2):
                pltpu.sync_copy(idx_hbm.at[f, pl.ds(cb * BCHUNK, BCHUNK)],
                                idxb)

                def gather16(i, carry3):
                    for u in range(UNROLL):
                        s = pl.ds((i * UNROLL + u) * _L, _L)
                        outb[s] = plsc.load_gather(slab, [idxb[s]])
                    return carry3

                lax.fori_loop(0, BCHUNK // (_L * UNROLL), gather16, 0)
                pltpu.sync_copy(
                    outb, out_hbm.at[pl.ds(row * B + cb * BCHUNK, BCHUNK)]
                )
                return carry2

            lax.fori_loop(0, n_chunks, do_chunk, 0)
            return carry

        lax.fori_loop(0, rows_per_w, do_row, 0)

    return gather_kernel


def kernel(inputs, tables):
    B, F = inputs.shape
    _, V, D = tables.shape
    tab2 = jnp.transpose(tables, (0, 2, 1)).reshape(F * D, V)
    idx2 = inputs.T
    out = _make_gather_dmajor(F, V, D, B)(tab2, idx2)
    return out.reshape(F, D, B).transpose(2, 0, 1)


# speedup vs baseline: 1.8170x; 1.1762x over previous
import functools

import jax
import jax.numpy as jnp
from jax import lax
from jax.experimental import pallas as pl
from jax.experimental.pallas import tpu as pltpu
from jax.experimental.pallas import tpu_sc as plsc

_info = plsc.get_sparse_core_info()
_NC, _NS, _L = _info.num_cores, _info.num_subcores, _info.num_lanes
_NW = _NC * _NS


def _make_gather_dmajor(F, V, D, B):
    R = F * D
    assert R % _NW == 0
    rows_per_w = R // _NW
    BCHUNK = 2048
    n_chunks = B // BCHUNK
    UNROLL = 8
    NSPLIT = 4
    VPART = V // NSPLIT
    mesh = plsc.VectorSubcoreMesh(core_axis_name="c", subcore_axis_name="s")

    @functools.partial(
        pl.kernel,
        mesh=mesh,
        out_type=jax.ShapeDtypeStruct((R * B,), jnp.float32),
        compiler_params=pltpu.CompilerParams(
            use_tc_tiling_on_sc=False, needs_layout_passes=False
        ),
        scratch_types=[
            pltpu.VMEM((V,), jnp.float32),
            pltpu.VMEM((B,), jnp.int32),
            pltpu.VMEM((BCHUNK,), jnp.float32),
            pltpu.VMEM((BCHUNK,), jnp.float32),
            pltpu.SemaphoreType.DMA,
            pltpu.SemaphoreType.DMA,
            pltpu.SemaphoreType.DMA,
            pltpu.SemaphoreType.DMA,
        ],
    )
    def gather_kernel(tab_hbm, idx_hbm, out_hbm, slab, idxb, ob0, ob1,
                      ssem, isem, osem0, osem1):
        wid = lax.axis_index("s") * _NC + lax.axis_index("c")
        row0 = wid * rows_per_w
        obs = [ob0, ob1]
        osems = [osem0, osem1]

        def do_row(t, carry):
            row = row0 + t
            f = row // D
            scps = [
                pltpu.async_copy(
                    tab_hbm.at[row, pl.ds(q * VPART, VPART)],
                    slab.at[pl.ds(q * VPART, VPART)],
                    ssem,
                )
                for q in range(NSPLIT)
            ]
            icp = pltpu.async_copy(idx_hbm.at[f], idxb, isem)
            for cp in scps:
                cp.wait()
            icp.wait()

            for cb in range(n_chunks):
                p = cb & 1
                ob = obs[p]
                if cb >= 2:
                    pltpu.make_async_copy(
                        ob,
                        out_hbm.at[pl.ds(row * B + (cb - 2) * BCHUNK, BCHUNK)],
                        osems[p],
                    ).wait()
                else:
                    @pl.when(t > 0)
                    def _():
                        pltpu.make_async_copy(
                            ob,
                            out_hbm.at[pl.ds(0, BCHUNK)],
                            osems[p],
                        ).wait()

                def gather16(i, carry3):
                    base = cb * BCHUNK
                    for u in range(UNROLL):
                        s = pl.ds((i * UNROLL + u) * _L, _L)
                        sb = pl.ds(base + (i * UNROLL + u) * _L, _L)
                        ob[s] = plsc.load_gather(slab, [idxb[sb]])
                    return carry3

                lax.fori_loop(0, BCHUNK // (_L * UNROLL), gather16, 0)
                pltpu.async_copy(
                    ob, out_hbm.at[pl.ds(row * B + cb * BCHUNK, BCHUNK)],
                    osems[p],
                )
            return carry

        lax.fori_loop(0, rows_per_w, do_row, 0)

        for p in range(2):
            pltpu.make_async_copy(
                obs[p], out_hbm.at[pl.ds(0, BCHUNK)], osems[p]
            ).wait()

    return gather_kernel


def kernel(inputs, tables):
    B, F = inputs.shape
    _, V, D = tables.shape
    tab2 = jnp.transpose(tables, (0, 2, 1)).reshape(F * D, V)
    idx2 = inputs.T
    out = _make_gather_dmajor(F, V, D, B)(tab2, idx2)
    return out.reshape(F, D, B).transpose(2, 0, 1)
